# Initial kernel scaffold; baseline (speedup 1.0000x reference)
import jax
import jax.numpy as jnp
from jax.experimental import pallas as pl


def kernel(x_cat, tables, W1, b1, W2, b2, W3, b3):
    raise NotImplementedError("write your pallas kernel here")


# speedup vs baseline: 8.0942x; 8.0942x over previous
import functools

import jax
import jax.numpy as jnp
from jax import lax
from jax.experimental import pallas as pl
from jax.experimental.pallas import tpu as pltpu
from jax.experimental.pallas import tpu_sc as plsc

B = 16384
F = 26
V = 100000
D = 32
P = 128

NC = 2
NS = 16
NW = NC * NS
ROWS = B * F
RPW = ROWS // NW
CHUNK = 128
CPW = RPW // CHUNK
K = 4
NGRP = CPW // K
HGRP = NGRP // 2

ROW_BYTES = D * 4
BUF_BYTES = CHUNK * ROW_BYTES


def _gather_body(gidx_hbm, tab_hbm, out_hbm, idx_v, bufs_a, bufs_b, sem_g, sem_s):
  cid = lax.axis_index("c")
  sid = lax.axis_index("s")
  wid = sid * NC + cid
  chunk0 = wid * CPW

  pltpu.sync_copy(gidx_hbm.at[pl.ds(chunk0, CPW)], idx_v)

  def fire_gather(c_local, buf):
    pltpu.async_copy(tab_hbm.at[idx_v.at[c_local]], buf, sem_g)

  def fire_store(c_local, buf):
    row0 = wid * RPW + c_local * CHUNK
    pltpu.async_copy(buf, out_hbm.at[pl.ds(row0, CHUNK)], sem_s)

  def drain(sem, buf):
    pltpu.make_async_copy(tab_hbm.at[idx_v.at[0]], buf, sem).wait()

  for j in range(K):
    fire_gather(j, bufs_a[j])

  def body(h, _):
    g0 = 2 * h
    g1 = g0 + 1
    for j in range(K):
      drain(sem_g, bufs_a[j])
    @pl.when(h > 0)
    def _():
      for j in range(K):
        drain(sem_s, bufs_b[j])
    for j in range(K):
      fire_gather(g1 * K + j, bufs_b[j])
    for j in range(K):
      fire_store(g0 * K + j, bufs_a[j])
    for j in range(K):
      drain(sem_g, bufs_b[j])
    for j in range(K):
      drain(sem_s, bufs_a[j])

    @pl.when(h < HGRP - 1)
    def _():
      for j in range(K):
        fire_gather((g1 + 1) * K + j, bufs_a[j])
    for j in range(K):
      fire_store(g1 * K + j, bufs_b[j])
    return 0

  lax.fori_loop(0, HGRP, body, 0)

  for j in range(K):
    drain(sem_s, bufs_b[j])


@functools.partial(
    pl.kernel,
    out_type=jax.ShapeDtypeStruct((ROWS, D), jnp.float32),
    mesh=plsc.VectorSubcoreMesh(core_axis_name="c", subcore_axis_name="s"),
    compiler_params=pltpu.CompilerParams(use_tc_tiling_on_sc=False),
    scratch_types=[
        pltpu.VMEM((CPW, CHUNK), jnp.int32),
        [pltpu.VMEM((CHUNK, D), jnp.float32) for _ in range(K)],
        [pltpu.VMEM((CHUNK, D), jnp.float32) for _ in range(K)],
        pltpu.SemaphoreType.DMA,
        pltpu.SemaphoreType.DMA,
    ],
)
def _sc_gather(gidx_hbm, tab_hbm, out_hbm, idx_v, bufs_a, bufs_b, sem_g, sem_s):
  _gather_body(gidx_hbm, tab_hbm, out_hbm, idx_v, bufs_a, bufs_b, sem_g, sem_s)


_SQRT_HALF = 0.7071067811865476


def _gelu(x):
  return x * 0.5 * (1.0 + lax.erf(x * _SQRT_HALF))


def _mlp_body(e_ref, w1_ref, b1_ref, w2_ref, b2_ref, w3_ref, b3_ref, o_ref):
  h = e_ref[...]
  h = _gelu(jnp.dot(h, w1_ref[...], preferred_element_type=jnp.float32)
            + b1_ref[...])
  h = _gelu(jnp.dot(h, w2_ref[...], preferred_element_type=jnp.float32)
            + b2_ref[...])
  o_ref[...] = (jnp.dot(h, w3_ref[...], preferred_element_type=jnp.float32)
                + b3_ref[...])


BLK = 1024


def _mlp(emb, W1, b1, W2, b2, W3, b3):
  grid = (B // BLK,)
  full = lambda shape: pl.BlockSpec(shape, lambda i: (0, 0))
  return pl.pallas_call(
      _mlp_body,
      grid=grid,
      in_specs=[
          pl.BlockSpec((BLK, F * D), lambda i: (i, 0)),
          full((F * D, P)),
          full((1, P)),
          full((P, P)),
          full((1, P)),
          full((P, P)),
          full((1, P)),
      ],
      out_specs=pl.BlockSpec((BLK, P), lambda i: (i, 0)),
      out_shape=jax.ShapeDtypeStruct((B, P), jnp.float32),
  )(emb, W1, b1, W2, b2, W3, b3)


def kernel(x_cat, tables, W1, b1, W2, b2, W3, b3):
  gidx = (x_cat + (jnp.arange(F, dtype=jnp.int32) * V)[None, :]).reshape(
      ROWS // CHUNK, CHUNK)
  tab = tables.reshape(F * V, D)
  emb = _sc_gather(gidx, tab)
  h = emb.reshape(B, F * D)
  return _mlp(h, W1, b1.reshape(1, P), W2, b2.reshape(1, P),
              W3, b3.reshape(1, P))
